# baseline (speedup 1.0000x reference)
import jax
import jax.numpy as jnp
from jax.experimental import pallas as pl
from jax.experimental.pallas import tpu as pltpu

_B = 1024
_S = 64
_D = 1024
_H = 2048
_H2 = _H // 2
_NE = 8
_R = 128
_T = _B // _R
_G = 2 * _NE
_NA = _T + _G - 1
_NB = _T + _NE - 1
_EPS = 1e-5


def _routing(key16):
    onehot16 = (key16[:, None] == jnp.arange(_G, dtype=jnp.int32)[None, :]
                ).astype(jnp.int32)
    cum = jnp.cumsum(onehot16, axis=0)
    counts16 = cum[-1]
    starts16 = jnp.cumsum(counts16) - counts16
    rank = jnp.sum(onehot16 * (cum - 1), axis=1)
    base = jnp.sum(onehot16 * starts16[None, :], axis=1)
    oinv = (base + rank).astype(jnp.int32)

    counts8 = counts16[0::2] + counts16[1::2]

    def tables(counts, ngroups, ii):
        starts = jnp.cumsum(counts) - counts
        ends = starts + counts
        tlo = starts // _R
        thi = (ends + _R - 1) // _R
        ntiles = jnp.where(counts > 0, thi - tlo, 0)
        iend = jnp.cumsum(ntiles)
        total = iend[ngroups - 1]
        g = jnp.searchsorted(iend, ii, side='right').astype(jnp.int32)
        valid = ii < total
        g = jnp.minimum(g, ngroups - 1)
        first = iend[g] - ntiles[g]
        t = tlo[g] + (ii - first)
        t = jnp.clip(jnp.where(valid, t, _T - 1), 0, _T - 1)
        lo = starts[g]
        hi = jnp.where(valid, ends[g], 0)
        return g, t, lo, hi, valid

    ii_a = jnp.arange(_NA, dtype=jnp.int32)
    ii_b = jnp.arange(_NB, dtype=jnp.int32)
    ga, ta, loa, hia, va = tables(counts16, _G, ii_a)
    gb, tb, lob, hib, vb = tables(counts8, _NE, ii_b)

    emb_a = ga // 2
    par_a = ga % 2

    def hold_last(use, e, ii):
        enc = jnp.where(use, ii * _NE + e, -1)
        run = jax.lax.cummax(enc)
        return jnp.where(run >= 0, run % _NE, 0).astype(jnp.int32)

    phe = hold_last((par_a == 0) & va, emb_a, ii_a)
    pre = hold_last((par_a == 1) & va, emb_a, ii_a)
    eb = hold_last(vb, gb, ii_b)

    to32 = lambda x: x.astype(jnp.int32)
    return (oinv,
            to32(ta), to32(loa), to32(hia), to32(par_a), to32(va),
            to32(emb_a), phe, pre,
            to32(tb), to32(lob), to32(hib), to32(vb), eb)


def _gelu(x):
    return 0.5 * x * (1.0 + jax.lax.erf(x * 0.7071067811865476))


def _bf(x):
    return x.astype(jnp.bfloat16)


def _stage_a_body(s_t, s_lo, s_hi, s_par, s_valid, s_emb, s_phe, s_pre,
                  oinv_ref, x_ref,
                  wph1a_ref, wph1b_ref, bph1_ref,
                  wph2a_ref, wph2b_ref, bph2_ref,
                  wpr1a_ref, wpr1b_ref, bpr1_ref,
                  wpr2a_ref, wpr2b_ref, bpr2_ref,
                  lng_ref, lnb_ref, out_ref):
    i = pl.program_id(0)
    valid = s_valid[i] > 0
    par = s_par[i]

    def run(w1a_ref, w1b_ref, b1_ref, w2a_ref, w2b_ref, b2_ref):
        p0 = s_t[i] * _R + jax.lax.broadcasted_iota(jnp.int32, (_R, 1), 0)
        onehot = (oinv_ref[...] == p0).astype(jnp.float32)
        xs = jnp.dot(onehot, x_ref[...], preferred_element_type=jnp.float32)
        xsb = _bf(xs)
        ha = _gelu(jnp.dot(xsb, _bf(w1a_ref[0]),
                           preferred_element_type=jnp.float32)
                   + b1_ref[0, :, :_H2])
        hb = _gelu(jnp.dot(xsb, _bf(w1b_ref[0]),
                           preferred_element_type=jnp.float32)
                   + b1_ref[0, :, _H2:])
        y = (jnp.dot(_bf(ha), _bf(w2a_ref[0]),
                     preferred_element_type=jnp.float32)
             + jnp.dot(_bf(hb), _bf(w2b_ref[0]),
                       preferred_element_type=jnp.float32)
             + b2_ref[0])
        mu = jnp.mean(y, axis=1, keepdims=True)
        var = jnp.mean(jnp.square(y - mu), axis=1, keepdims=True)
        yn = (y - mu) * jax.lax.rsqrt(var + _EPS) * lng_ref[0] + lnb_ref[0]
        gmask = (p0 >= s_lo[i]) & (p0 < s_hi[i])
        out_ref[...] = jnp.where(gmask, yn, out_ref[...])

    @pl.when(valid & (par == 0))
    def _():
        run(wph1a_ref, wph1b_ref, bph1_ref, wph2a_ref, wph2b_ref, bph2_ref)

    @pl.when(valid & (par == 1))
    def _():
        run(wpr1a_ref, wpr1b_ref, bpr1_ref, wpr2a_ref, wpr2b_ref, bpr2_ref)


def _stage_b_body(s_t, s_lo, s_hi, s_valid, s_e,
                  oinvc_ref, xin_ref,
                  wt1a_ref, wt1b_ref, bt1_ref,
                  wt2a_ref, wt2b_ref, bt2_ref,
                  out_ref):
    i = pl.program_id(0)

    @pl.when(i == 0)
    def _():
        out_ref[...] = jnp.zeros_like(out_ref)

    @pl.when(s_valid[i] > 0)
    def _():
        xsb = _bf(xin_ref[...])
        ha = _gelu(jnp.dot(xsb, _bf(wt1a_ref[0]),
                           preferred_element_type=jnp.float32)
                   + bt1_ref[0, :, :_H2])
        hb = _gelu(jnp.dot(xsb, _bf(wt1b_ref[0]),
                           preferred_element_type=jnp.float32)
                   + bt1_ref[0, :, _H2:])
        y = (jnp.dot(_bf(ha), _bf(wt2a_ref[0]),
                     preferred_element_type=jnp.float32)
             + jnp.dot(_bf(hb), _bf(wt2b_ref[0]),
                       preferred_element_type=jnp.float32)
             + bt2_ref[0])
        p0 = s_t[i] * _R + jax.lax.broadcasted_iota(jnp.int32, (_R, 1), 0)
        ym = jnp.where((p0 >= s_lo[i]) & (p0 < s_hi[i]), y, 0.0)
        p1 = s_t[i] * _R + jax.lax.broadcasted_iota(jnp.int32, (1, _R), 1)
        scat = (oinvc_ref[...] == p1).astype(jnp.float32)
        out_ref[...] += jnp.dot(scat, ym, preferred_element_type=jnp.float32)


@jax.jit
def kernel(raw_state, has_proprio, embodiment_idx, W_ph1, b_ph1, W_ph2, b_ph2,
           W_pr1, b_pr1, W_pr2, b_pr2, ln_g, ln_b, W_t1, b_t1, W_t2, b_t2):
    key16 = (embodiment_idx.astype(jnp.int32) * 2
             + has_proprio.astype(jnp.int32))
    (oinv, ta, loa, hia, para, va, emba, phe, pre,
     tb, lob, hib, vb, eb) = _routing(key16)

    oinv_row = oinv.reshape(1, _B)
    oinv_col = oinv.reshape(_B, 1)

    def amap(*path):
        def f(i, st, slo, shi, sp, sv, se, sphe, spre):
            refs = {'t': st, 'phe': sphe, 'pre': spre, 'e': se}
            return tuple(refs[p][i] if isinstance(p, str) else p for p in path)
        return f

    mixed_ln = pl.pallas_call(
        _stage_a_body,
        grid_spec=pltpu.PrefetchScalarGridSpec(
            num_scalar_prefetch=8,
            grid=(_NA,),
            in_specs=[
                pl.BlockSpec((1, _B), amap(0, 0)),
                pl.BlockSpec((_B, _S), amap(0, 0)),
                pl.BlockSpec((1, _S, _H2), amap('phe', 0, 0)),
                pl.BlockSpec((1, _S, _H2), amap('phe', 0, 1)),
                pl.BlockSpec((1, 1, _H), amap('phe', 0, 0)),
                pl.BlockSpec((1, _H2, _D), amap('phe', 0, 0)),
                pl.BlockSpec((1, _H2, _D), amap('phe', 1, 0)),
                pl.BlockSpec((1, 1, _D), amap('phe', 0, 0)),
                pl.BlockSpec((1, _S, _H2), amap('pre', 0, 0)),
                pl.BlockSpec((1, _S, _H2), amap('pre', 0, 1)),
                pl.BlockSpec((1, 1, _H), amap('pre', 0, 0)),
                pl.BlockSpec((1, _H2, _D), amap('pre', 0, 0)),
                pl.BlockSpec((1, _H2, _D), amap('pre', 1, 0)),
                pl.BlockSpec((1, 1, _D), amap('pre', 0, 0)),
                pl.BlockSpec((1, 1, _D), amap('e', 0, 0)),
                pl.BlockSpec((1, 1, _D), amap('e', 0, 0)),
            ],
            out_specs=pl.BlockSpec((_R, _D), amap('t', 0)),
        ),
        out_shape=jax.ShapeDtypeStruct((_B, _D), jnp.float32),
    )(ta, loa, hia, para, va, emba, phe, pre,
      oinv_row, raw_state,
      W_ph1, W_ph1, b_ph1[:, None, :], W_ph2, W_ph2, b_ph2[:, None, :],
      W_pr1, W_pr1, b_pr1[:, None, :], W_pr2, W_pr2, b_pr2[:, None, :],
      ln_g[:, None, :], ln_b[:, None, :])

    def bmap(*path):
        def f(i, st, slo, shi, sv, se):
            refs = {'t': st, 'e': se}
            return tuple(refs[p][i] if isinstance(p, str) else p for p in path)
        return f

    out = pl.pallas_call(
        _stage_b_body,
        grid_spec=pltpu.PrefetchScalarGridSpec(
            num_scalar_prefetch=5,
            grid=(_NB,),
            in_specs=[
                pl.BlockSpec((_B, 1), bmap(0, 0)),
                pl.BlockSpec((_R, _D), bmap('t', 0)),
                pl.BlockSpec((1, _D, _H2), bmap('e', 0, 0)),
                pl.BlockSpec((1, _D, _H2), bmap('e', 0, 1)),
                pl.BlockSpec((1, 1, _H), bmap('e', 0, 0)),
                pl.BlockSpec((1, _H2, _D), bmap('e', 0, 0)),
                pl.BlockSpec((1, _H2, _D), bmap('e', 1, 0)),
                pl.BlockSpec((1, 1, _D), bmap('e', 0, 0)),
            ],
            out_specs=pl.BlockSpec((_B, _D), bmap(0, 0)),
        ),
        out_shape=jax.ShapeDtypeStruct((_B, _D), jnp.float32),
    )(tb, lob, hib, vb, eb,
      oinv_col, mixed_ln,
      W_t1, W_t1, b_t1[:, None, :], W_t2, W_t2, b_t2[:, None, :])

    return out[:, None, :]


# speedup vs baseline: 1.2841x; 1.2841x over previous
import jax
import jax.numpy as jnp
from jax.experimental import pallas as pl
from jax.experimental.pallas import tpu as pltpu

_B = 1024
_S = 64
_D = 1024
_H = 2048
_NE = 8
_R = 128
_T = _B // _R
_G = 2 * _NE
_EPS = 1e-5
_NC = 4

_f32 = jnp.float32


def _routing(key16):
    onehot16 = (key16[:, None] == jnp.arange(_G, dtype=jnp.int32)[None, :]
                ).astype(jnp.int32)
    cum = jnp.cumsum(onehot16, axis=0)
    counts16 = cum[-1]
    starts16 = jnp.cumsum(counts16) - counts16
    rank = jnp.sum(onehot16 * (cum - 1), axis=1)
    base = jnp.sum(onehot16 * starts16[None, :], axis=1)
    oinv = (base + rank).astype(jnp.int32)

    def tables(counts):
        starts = jnp.cumsum(counts) - counts
        ends = starts + counts
        tlo = starts // _R
        thi = (ends + _R - 1) // _R
        ntl = jnp.where(counts > 0, thi - tlo, 0)
        return (tlo.astype(jnp.int32), ntl.astype(jnp.int32),
                starts.astype(jnp.int32), ends.astype(jnp.int32))

    counts8 = counts16[0::2] + counts16[1::2]
    return oinv, tables(counts16), tables(counts8)


def _gelu(x):
    return 0.5 * x * (1.0 + jax.lax.erf(x * 0.7071067811865476))


def _bf(x):
    return x.astype(jnp.bfloat16)


def _copy_expert(hbm_ref, slots_ref, sems_ref, k, start):
    slot = jax.lax.rem(k, 2)
    rows = hbm_ref.shape[1]
    c_rows = rows // _NC
    for c in range(_NC):
        cp = pltpu.make_async_copy(
            hbm_ref.at[k, pl.ds(c * c_rows, c_rows), :],
            slots_ref.at[slot, pl.ds(c * c_rows, c_rows), :],
            sems_ref.at[slot, c])
        if start:
            cp.start()
        else:
            cp.wait()


def _stage_a_body(s_tlo, s_ntl, s_lo, s_hi,
                  oinv_ref, x_ref, wph1_ref, wpr1_ref,
                  bph1_ref, bph2_ref, bpr1_ref, bpr2_ref,
                  lng_ref, lnb_ref,
                  wph2_hbm, wpr2_hbm,
                  out_ref,
                  ph_slots, pr_slots, ph_sems, pr_sems):
    g = pl.program_id(0)
    f = jax.lax.rem(g, 2)
    k = g // 2

    @pl.when(g == 0)
    def _():
        _copy_expert(wph2_hbm, ph_slots, ph_sems, 0, True)
        _copy_expert(wpr2_hbm, pr_slots, pr_sems, 0, True)

    @pl.when((f == 0) & (k + 1 < _NE))
    def _():
        _copy_expert(wph2_hbm, ph_slots, ph_sems, k + 1, True)

    @pl.when((f == 1) & (k + 1 < _NE))
    def _():
        _copy_expert(wpr2_hbm, pr_slots, pr_sems, k + 1, True)

    def run(w1_ref, b1_ref, w2_hbm, slots, sems, b2_ref):
        _copy_expert(w2_hbm, slots, sems, k, False)
        slot = jax.lax.rem(k, 2)
        w1b = _bf(w1_ref[k])
        w2b = _bf(slots[slot])
        b1 = b1_ref[k]
        b2 = b2_ref[k]
        lng = lng_ref[k]
        lnb = lnb_ref[k]
        lo = s_lo[g]
        hi = s_hi[g]

        def tile(it, _):
            t = s_tlo[g] + it
            p0 = t * _R + jax.lax.broadcasted_iota(jnp.int32, (_R, 1), 0)
            onehot = (oinv_ref[...] == p0).astype(_f32)
            xs = jnp.dot(onehot, x_ref[...], preferred_element_type=_f32)
            h = _gelu(jnp.dot(_bf(xs), w1b, preferred_element_type=_f32) + b1)
            y = jnp.dot(_bf(h), w2b, preferred_element_type=_f32) + b2
            mu = jnp.mean(y, axis=1, keepdims=True)
            var = jnp.mean(jnp.square(y - mu), axis=1, keepdims=True)
            yn = (y - mu) * jax.lax.rsqrt(var + _EPS) * lng + lnb
            gmask = (p0 >= lo) & (p0 < hi)
            out_ref[pl.ds(t * _R, _R), :] = jnp.where(
                gmask, yn, out_ref[pl.ds(t * _R, _R), :])
            return 0

        jax.lax.fori_loop(0, s_ntl[g], tile, 0)

    @pl.when(f == 0)
    def _():
        run(wph1_ref, bph1_ref, wph2_hbm, ph_slots, ph_sems, bph2_ref)

    @pl.when(f == 1)
    def _():
        run(wpr1_ref, bpr1_ref, wpr2_hbm, pr_slots, pr_sems, bpr2_ref)


def _stage_b_body(s_tlo, s_ntl, s_lo, s_hi,
                  oinvc_ref, xin_ref, bt1_ref, bt2_ref,
                  wt1_hbm, wt2_hbm,
                  out_ref,
                  t1_slots, t2_slots, t1_sems, t2_sems):
    e = pl.program_id(0)

    @pl.when(e == 0)
    def _():
        _copy_expert(wt1_hbm, t1_slots, t1_sems, 0, True)
        _copy_expert(wt2_hbm, t2_slots, t2_sems, 0, True)
        out_ref[...] = jnp.zeros_like(out_ref)

    @pl.when(e + 1 < _NE)
    def _():
        _copy_expert(wt1_hbm, t1_slots, t1_sems, e + 1, True)
        _copy_expert(wt2_hbm, t2_slots, t2_sems, e + 1, True)

    _copy_expert(wt1_hbm, t1_slots, t1_sems, e, False)
    _copy_expert(wt2_hbm, t2_slots, t2_sems, e, False)
    slot = jax.lax.rem(e, 2)
    w1b = _bf(t1_slots[slot])
    w2b = _bf(t2_slots[slot])
    b1 = bt1_ref[e]
    b2 = bt2_ref[e]
    lo = s_lo[e]
    hi = s_hi[e]

    def tile(it, _):
        t = s_tlo[e] + it
        xs = xin_ref[pl.ds(t * _R, _R), :]
        h = _gelu(jnp.dot(_bf(xs), w1b, preferred_element_type=_f32) + b1)
        y = jnp.dot(_bf(h), w2b, preferred_element_type=_f32) + b2
        p0 = t * _R + jax.lax.broadcasted_iota(jnp.int32, (_R, 1), 0)
        ym = jnp.where((p0 >= lo) & (p0 < hi), y, 0.0)
        p1 = t * _R + jax.lax.broadcasted_iota(jnp.int32, (1, _R), 1)
        scat = (oinvc_ref[...] == p1).astype(_f32)
        out_ref[...] += jnp.dot(scat, ym, preferred_element_type=_f32)
        return 0

    jax.lax.fori_loop(0, s_ntl[e], tile, 0)


_VMEM_FULL = lambda: pl.BlockSpec(memory_space=pltpu.MemorySpace.VMEM)
_HBM = lambda: pl.BlockSpec(memory_space=pltpu.MemorySpace.HBM)


@jax.jit
def kernel(raw_state, has_proprio, embodiment_idx, W_ph1, b_ph1, W_ph2, b_ph2,
           W_pr1, b_pr1, W_pr2, b_pr2, ln_g, ln_b, W_t1, b_t1, W_t2, b_t2):
    key16 = (embodiment_idx.astype(jnp.int32) * 2
             + has_proprio.astype(jnp.int32))
    oinv, (tlo16, ntl16, lo16, hi16), (tlo8, ntl8, lo8, hi8) = _routing(key16)

    mixed_ln = pl.pallas_call(
        _stage_a_body,
        grid_spec=pltpu.PrefetchScalarGridSpec(
            num_scalar_prefetch=4,
            grid=(_G,),
            in_specs=[_VMEM_FULL() for _ in range(10)] + [_HBM(), _HBM()],
            out_specs=_VMEM_FULL(),
            scratch_shapes=[
                pltpu.VMEM((2, _H, _D), _f32),
                pltpu.VMEM((2, _H, _D), _f32),
                pltpu.SemaphoreType.DMA((2, _NC)),
                pltpu.SemaphoreType.DMA((2, _NC)),
            ],
        ),
        out_shape=jax.ShapeDtypeStruct((_B, _D), _f32),
        compiler_params=pltpu.CompilerParams(
            dimension_semantics=("arbitrary",),
            vmem_limit_bytes=120 * 1024 * 1024,
        ),
    )(tlo16, ntl16, lo16, hi16,
      oinv.reshape(1, _B), raw_state, W_ph1, W_pr1,
      b_ph1[:, None, :], b_ph2[:, None, :],
      b_pr1[:, None, :], b_pr2[:, None, :],
      ln_g[:, None, :], ln_b[:, None, :],
      W_ph2, W_pr2)

    out = pl.pallas_call(
        _stage_b_body,
        grid_spec=pltpu.PrefetchScalarGridSpec(
            num_scalar_prefetch=4,
            grid=(_NE,),
            in_specs=[_VMEM_FULL() for _ in range(4)] + [_HBM(), _HBM()],
            out_specs=_VMEM_FULL(),
            scratch_shapes=[
                pltpu.VMEM((2, _D, _H), _f32),
                pltpu.VMEM((2, _H, _D), _f32),
                pltpu.SemaphoreType.DMA((2, _NC)),
                pltpu.SemaphoreType.DMA((2, _NC)),
            ],
        ),
        out_shape=jax.ShapeDtypeStruct((_B, _D), _f32),
        compiler_params=pltpu.CompilerParams(
            dimension_semantics=("arbitrary",),
            vmem_limit_bytes=120 * 1024 * 1024,
        ),
    )(tlo8, ntl8, lo8, hi8,
      oinv.reshape(_B, 1), mixed_ln,
      b_t1[:, None, :], b_t2[:, None, :],
      W_t1, W_t2)

    return out[:, None, :]


# speedup vs baseline: 1.3186x; 1.0268x over previous
import jax
import jax.numpy as jnp
from jax.experimental import pallas as pl
from jax.experimental.pallas import tpu as pltpu

_B = 1024
_S = 64
_D = 1024
_H = 2048
_NE = 8
_R = 128
_T = _B // _R
_G = 2 * _NE
_EPS = 1e-5
_NC = 8

_f32 = jnp.float32


def _routing(key16):
    onehot16 = (key16[:, None] == jnp.arange(_G, dtype=jnp.int32)[None, :]
                ).astype(jnp.int32)
    cum = jnp.cumsum(onehot16, axis=0)
    counts16 = cum[-1]
    starts16 = jnp.cumsum(counts16) - counts16
    rank = jnp.sum(onehot16 * (cum - 1), axis=1)
    base = jnp.sum(onehot16 * starts16[None, :], axis=1)
    oinv = (base + rank).astype(jnp.int32)

    def tables(counts):
        starts = jnp.cumsum(counts) - counts
        ends = starts + counts
        tlo = starts // _R
        thi = (ends + _R - 1) // _R
        ntl = jnp.where(counts > 0, thi - tlo, 0)
        return (tlo.astype(jnp.int32), ntl.astype(jnp.int32),
                starts.astype(jnp.int32), ends.astype(jnp.int32))

    counts8 = counts16[0::2] + counts16[1::2]
    return oinv, tables(counts16), tables(counts8)


def _gelu(x):
    return 0.5 * x * (1.0 + jax.lax.erf(x * 0.7071067811865476))


def _bf(x):
    return x.astype(jnp.bfloat16)


def _copy_expert(hbm_ref, slots_ref, sems_ref, k, start):
    slot = jax.lax.rem(k, 2)
    rows = hbm_ref.shape[1]
    c_rows = rows // _NC
    for c in range(_NC):
        cp = pltpu.make_async_copy(
            hbm_ref.at[k, pl.ds(c * c_rows, c_rows), :],
            slots_ref.at[slot, pl.ds(c * c_rows, c_rows), :],
            sems_ref.at[slot, c])
        if start:
            cp.start()
        else:
            cp.wait()


def _stage_a_body(s_tlo, s_ntl, s_lo, s_hi,
                  oinv_ref, x_ref, wph1_ref, wpr1_ref,
                  bph1_ref, bph2_ref, bpr1_ref, bpr2_ref,
                  lng_ref, lnb_ref,
                  wph2_hbm, wpr2_hbm,
                  out_ref,
                  xs_all, ph_slots, pr_slots, ph_sems, pr_sems):
    g = pl.program_id(0)
    f = jax.lax.rem(g, 2)
    k = g // 2

    @pl.when(g == 0)
    def _():
        _copy_expert(wph2_hbm, ph_slots, ph_sems, 0, True)
        _copy_expert(wpr2_hbm, pr_slots, pr_sems, 0, True)
        gat = (oinv_ref[...] ==
               jax.lax.broadcasted_iota(jnp.int32, (_B, 1), 0)).astype(_f32)
        xs_all[...] = jnp.dot(gat, x_ref[...], preferred_element_type=_f32)

    @pl.when((f == 0) & (k + 1 < _NE))
    def _():
        _copy_expert(wph2_hbm, ph_slots, ph_sems, k + 1, True)

    @pl.when((f == 1) & (k + 1 < _NE))
    def _():
        _copy_expert(wpr2_hbm, pr_slots, pr_sems, k + 1, True)

    def run(w1_ref, b1_ref, w2_hbm, slots, sems, b2_ref):
        _copy_expert(w2_hbm, slots, sems, k, False)
        slot = jax.lax.rem(k, 2)
        w1b = _bf(w1_ref[k])
        w2b = _bf(slots[slot])
        b1 = b1_ref[k]
        b2 = b2_ref[k]
        lng = lng_ref[k]
        lnb = lnb_ref[k]
        lo = s_lo[g]
        hi = s_hi[g]

        def tile(it, _):
            t = s_tlo[g] + it
            p0 = t * _R + jax.lax.broadcasted_iota(jnp.int32, (_R, 1), 0)
            xs = xs_all[pl.ds(t * _R, _R), :]
            h = _gelu(jnp.dot(_bf(xs), w1b, preferred_element_type=_f32) + b1)
            y = jnp.dot(_bf(h), w2b, preferred_element_type=_f32) + b2
            mu = jnp.mean(y, axis=1, keepdims=True)
            var = jnp.mean(jnp.square(y - mu), axis=1, keepdims=True)
            yn = (y - mu) * jax.lax.rsqrt(var + _EPS) * lng + lnb
            gmask = (p0 >= lo) & (p0 < hi)
            out_ref[pl.ds(t * _R, _R), :] = jnp.where(
                gmask, yn, out_ref[pl.ds(t * _R, _R), :])
            return 0

        jax.lax.fori_loop(0, s_ntl[g], tile, 0)

    @pl.when(f == 0)
    def _():
        run(wph1_ref, bph1_ref, wph2_hbm, ph_slots, ph_sems, bph2_ref)

    @pl.when(f == 1)
    def _():
        run(wpr1_ref, bpr1_ref, wpr2_hbm, pr_slots, pr_sems, bpr2_ref)


def _stage_b_body(s_tlo, s_ntl, s_lo, s_hi,
                  oinvc_ref, xin_ref, bt1_ref, bt2_ref,
                  wt1_hbm, wt2_hbm,
                  out_ref,
                  ysort, t1_slots, t2_slots, t1_sems, t2_sems):
    e = pl.program_id(0)

    @pl.when(e == 0)
    def _():
        _copy_expert(wt1_hbm, t1_slots, t1_sems, 0, True)
        _copy_expert(wt2_hbm, t2_slots, t2_sems, 0, True)

    @pl.when(e + 1 < _NE)
    def _():
        _copy_expert(wt1_hbm, t1_slots, t1_sems, e + 1, True)
        _copy_expert(wt2_hbm, t2_slots, t2_sems, e + 1, True)

    _copy_expert(wt1_hbm, t1_slots, t1_sems, e, False)
    _copy_expert(wt2_hbm, t2_slots, t2_sems, e, False)
    slot = jax.lax.rem(e, 2)
    w1b = _bf(t1_slots[slot])
    w2b = _bf(t2_slots[slot])
    b1 = bt1_ref[e]
    b2 = bt2_ref[e]
    lo = s_lo[e]
    hi = s_hi[e]

    def tile(it, _):
        t = s_tlo[e] + it
        xs = xin_ref[pl.ds(t * _R, _R), :]
        h = _gelu(jnp.dot(_bf(xs), w1b, preferred_element_type=_f32) + b1)
        y = jnp.dot(_bf(h), w2b, preferred_element_type=_f32) + b2
        p0 = t * _R + jax.lax.broadcasted_iota(jnp.int32, (_R, 1), 0)
        gmask = (p0 >= lo) & (p0 < hi)
        ysort[pl.ds(t * _R, _R), :] = jnp.where(
            gmask, y, ysort[pl.ds(t * _R, _R), :])
        return 0

    jax.lax.fori_loop(0, s_ntl[e], tile, 0)

    @pl.when(e == _NE - 1)
    def _():
        scat = (oinvc_ref[...] ==
                jax.lax.broadcasted_iota(jnp.int32, (1, _B), 1)).astype(_f32)
        out_ref[...] = jnp.dot(scat, ysort[...], preferred_element_type=_f32)


_VMEM_FULL = lambda: pl.BlockSpec(memory_space=pltpu.MemorySpace.VMEM)
_HBM = lambda: pl.BlockSpec(memory_space=pltpu.MemorySpace.HBM)


@jax.jit
def kernel(raw_state, has_proprio, embodiment_idx, W_ph1, b_ph1, W_ph2, b_ph2,
           W_pr1, b_pr1, W_pr2, b_pr2, ln_g, ln_b, W_t1, b_t1, W_t2, b_t2):
    key16 = (embodiment_idx.astype(jnp.int32) * 2
             + has_proprio.astype(jnp.int32))
    oinv, (tlo16, ntl16, lo16, hi16), (tlo8, ntl8, lo8, hi8) = _routing(key16)

    mixed_ln = pl.pallas_call(
        _stage_a_body,
        grid_spec=pltpu.PrefetchScalarGridSpec(
            num_scalar_prefetch=4,
            grid=(_G,),
            in_specs=[_VMEM_FULL() for _ in range(10)] + [_HBM(), _HBM()],
            out_specs=_VMEM_FULL(),
            scratch_shapes=[
                pltpu.VMEM((_B, _S), _f32),
                pltpu.VMEM((2, _H, _D), _f32),
                pltpu.VMEM((2, _H, _D), _f32),
                pltpu.SemaphoreType.DMA((2, _NC)),
                pltpu.SemaphoreType.DMA((2, _NC)),
            ],
        ),
        out_shape=jax.ShapeDtypeStruct((_B, _D), _f32),
        compiler_params=pltpu.CompilerParams(
            dimension_semantics=("arbitrary",),
            vmem_limit_bytes=120 * 1024 * 1024,
        ),
    )(tlo16, ntl16, lo16, hi16,
      oinv.reshape(1, _B), raw_state, W_ph1, W_pr1,
      b_ph1[:, None, :], b_ph2[:, None, :],
      b_pr1[:, None, :], b_pr2[:, None, :],
      ln_g[:, None, :], ln_b[:, None, :],
      W_ph2, W_pr2)

    out = pl.pallas_call(
        _stage_b_body,
        grid_spec=pltpu.PrefetchScalarGridSpec(
            num_scalar_prefetch=4,
            grid=(_NE,),
            in_specs=[_VMEM_FULL() for _ in range(4)] + [_HBM(), _HBM()],
            out_specs=_VMEM_FULL(),
            scratch_shapes=[
                pltpu.VMEM((_B, _D), _f32),
                pltpu.VMEM((2, _D, _H), _f32),
                pltpu.VMEM((2, _H, _D), _f32),
                pltpu.SemaphoreType.DMA((2, _NC)),
                pltpu.SemaphoreType.DMA((2, _NC)),
            ],
        ),
        out_shape=jax.ShapeDtypeStruct((_B, _D), _f32),
        compiler_params=pltpu.CompilerParams(
            dimension_semantics=("arbitrary",),
            vmem_limit_bytes=120 * 1024 * 1024,
        ),
    )(tlo8, ntl8, lo8, hi8,
      oinv.reshape(_B, 1), mixed_ln,
      b_t1[:, None, :], b_t2[:, None, :],
      W_t1, W_t2)

    return out[:, None, :]


# speedup vs baseline: 1.3229x; 1.0033x over previous
import jax
import jax.numpy as jnp
from jax.experimental import pallas as pl
from jax.experimental.pallas import tpu as pltpu

_B = 1024
_S = 64
_D = 1024
_H = 2048
_NE = 8
_R = 128
_T = _B // _R
_G = 2 * _NE
_EPS = 1e-5
_NC = 16

_f32 = jnp.float32


def _routing(key16):
    onehot16 = (key16[:, None] == jnp.arange(_G, dtype=jnp.int32)[None, :]
                ).astype(jnp.int32)
    cum = jnp.cumsum(onehot16, axis=0)
    counts16 = cum[-1]
    starts16 = jnp.cumsum(counts16) - counts16
    rank = jnp.sum(onehot16 * (cum - 1), axis=1)
    base = jnp.sum(onehot16 * starts16[None, :], axis=1)
    oinv = (base + rank).astype(jnp.int32)

    def tables(counts):
        starts = jnp.cumsum(counts) - counts
        ends = starts + counts
        tlo = starts // _R
        thi = (ends + _R - 1) // _R
        ntl = jnp.where(counts > 0, thi - tlo, 0)
        return (tlo.astype(jnp.int32), ntl.astype(jnp.int32),
                starts.astype(jnp.int32), ends.astype(jnp.int32))

    counts8 = counts16[0::2] + counts16[1::2]
    return oinv, tables(counts16), tables(counts8)


def _gelu(x):
    return 0.5 * x * (1.0 + jax.lax.erf(x * 0.7071067811865476))


def _bf(x):
    return x.astype(jnp.bfloat16)


def _copy_expert(hbm_ref, slots_ref, sems_ref, k, start):
    slot = jax.lax.rem(k, 2)
    rows = hbm_ref.shape[1]
    c_rows = rows // _NC
    for c in range(_NC):
        cp = pltpu.make_async_copy(
            hbm_ref.at[k, pl.ds(c * c_rows, c_rows), :],
            slots_ref.at[slot, pl.ds(c * c_rows, c_rows), :],
            sems_ref.at[slot, c])
        if start:
            cp.start()
        else:
            cp.wait()


def _stage_a_body(s_tlo, s_ntl, s_lo, s_hi,
                  oinv_ref, x_ref, wph1_ref, wpr1_ref,
                  bph1_ref, bph2_ref, bpr1_ref, bpr2_ref,
                  lng_ref, lnb_ref,
                  wph2_hbm, wpr2_hbm,
                  out_ref,
                  xs_all, ph_slots, pr_slots, ph_sems, pr_sems):
    g = pl.program_id(0)
    f = jax.lax.rem(g, 2)
    k = g // 2

    @pl.when(g == 0)
    def _():
        _copy_expert(wph2_hbm, ph_slots, ph_sems, 0, True)
        _copy_expert(wpr2_hbm, pr_slots, pr_sems, 0, True)
        gat = (oinv_ref[...] ==
               jax.lax.broadcasted_iota(jnp.int32, (_B, 1), 0)).astype(_f32)
        xs_all[...] = jnp.dot(gat, x_ref[...], preferred_element_type=_f32)

    @pl.when((f == 0) & (k + 1 < _NE))
    def _():
        _copy_expert(wph2_hbm, ph_slots, ph_sems, k + 1, True)

    @pl.when((f == 1) & (k + 1 < _NE))
    def _():
        _copy_expert(wpr2_hbm, pr_slots, pr_sems, k + 1, True)

    def run(w1_ref, b1_ref, w2_hbm, slots, sems, b2_ref):
        _copy_expert(w2_hbm, slots, sems, k, False)
        slot = jax.lax.rem(k, 2)
        w1b = _bf(w1_ref[k])
        w2b = _bf(slots[slot])
        b1 = b1_ref[k]
        b2 = b2_ref[k]
        lng = lng_ref[k]
        lnb = lnb_ref[k]
        lo = s_lo[g]
        hi = s_hi[g]

        def tile(it, _):
            t = s_tlo[g] + it
            p0 = t * _R + jax.lax.broadcasted_iota(jnp.int32, (_R, 1), 0)
            xs = xs_all[pl.ds(t * _R, _R), :]
            h = _gelu(jnp.dot(_bf(xs), w1b, preferred_element_type=_f32) + b1)
            y = jnp.dot(_bf(h), w2b, preferred_element_type=_f32) + b2
            mu = jnp.mean(y, axis=1, keepdims=True)
            var = jnp.mean(jnp.square(y - mu), axis=1, keepdims=True)
            yn = (y - mu) * jax.lax.rsqrt(var + _EPS) * lng + lnb
            gmask = (p0 >= lo) & (p0 < hi)
            out_ref[pl.ds(t * _R, _R), :] = jnp.where(
                gmask, yn, out_ref[pl.ds(t * _R, _R), :])
            return 0

        jax.lax.fori_loop(0, s_ntl[g], tile, 0)

    @pl.when(f == 0)
    def _():
        run(wph1_ref, bph1_ref, wph2_hbm, ph_slots, ph_sems, bph2_ref)

    @pl.when(f == 1)
    def _():
        run(wpr1_ref, bpr1_ref, wpr2_hbm, pr_slots, pr_sems, bpr2_ref)


def _stage_b_body(s_tlo, s_ntl, s_lo, s_hi,
                  oinvc_ref, xin_ref, bt1_ref, bt2_ref,
                  wt1_hbm, wt2_hbm,
                  out_ref,
                  ysort, t1_slots, t2_slots, t1_sems, t2_sems):
    e = pl.program_id(0)

    @pl.when(e == 0)
    def _():
        _copy_expert(wt1_hbm, t1_slots, t1_sems, 0, True)
        _copy_expert(wt2_hbm, t2_slots, t2_sems, 0, True)

    @pl.when(e + 1 < _NE)
    def _():
        _copy_expert(wt1_hbm, t1_slots, t1_sems, e + 1, True)
        _copy_expert(wt2_hbm, t2_slots, t2_sems, e + 1, True)

    _copy_expert(wt1_hbm, t1_slots, t1_sems, e, False)
    _copy_expert(wt2_hbm, t2_slots, t2_sems, e, False)
    slot = jax.lax.rem(e, 2)
    w1b = _bf(t1_slots[slot])
    w2b = _bf(t2_slots[slot])
    b1 = bt1_ref[e]
    b2 = bt2_ref[e]
    lo = s_lo[e]
    hi = s_hi[e]

    def tile(it, _):
        t = s_tlo[e] + it
        xs = xin_ref[pl.ds(t * _R, _R), :]
        h = _gelu(jnp.dot(_bf(xs), w1b, preferred_element_type=_f32) + b1)
        y = jnp.dot(_bf(h), w2b, preferred_element_type=_f32) + b2
        p0 = t * _R + jax.lax.broadcasted_iota(jnp.int32, (_R, 1), 0)
        gmask = (p0 >= lo) & (p0 < hi)
        ysort[pl.ds(t * _R, _R), :] = jnp.where(
            gmask, y, ysort[pl.ds(t * _R, _R), :])
        return 0

    jax.lax.fori_loop(0, s_ntl[e], tile, 0)

    @pl.when(e == _NE - 1)
    def _():
        scat = (oinvc_ref[...] ==
                jax.lax.broadcasted_iota(jnp.int32, (1, _B), 1)).astype(_f32)
        out_ref[...] = jnp.dot(scat, ysort[...], preferred_element_type=_f32)


_VMEM_FULL = lambda: pl.BlockSpec(memory_space=pltpu.MemorySpace.VMEM)
_HBM = lambda: pl.BlockSpec(memory_space=pltpu.MemorySpace.HBM)


@jax.jit
def kernel(raw_state, has_proprio, embodiment_idx, W_ph1, b_ph1, W_ph2, b_ph2,
           W_pr1, b_pr1, W_pr2, b_pr2, ln_g, ln_b, W_t1, b_t1, W_t2, b_t2):
    key16 = (embodiment_idx.astype(jnp.int32) * 2
             + has_proprio.astype(jnp.int32))
    oinv, (tlo16, ntl16, lo16, hi16), (tlo8, ntl8, lo8, hi8) = _routing(key16)

    mixed_ln = pl.pallas_call(
        _stage_a_body,
        grid_spec=pltpu.PrefetchScalarGridSpec(
            num_scalar_prefetch=4,
            grid=(_G,),
            in_specs=[_VMEM_FULL() for _ in range(10)] + [_HBM(), _HBM()],
            out_specs=_VMEM_FULL(),
            scratch_shapes=[
                pltpu.VMEM((_B, _S), _f32),
                pltpu.VMEM((2, _H, _D), _f32),
                pltpu.VMEM((2, _H, _D), _f32),
                pltpu.SemaphoreType.DMA((2, _NC)),
                pltpu.SemaphoreType.DMA((2, _NC)),
            ],
        ),
        out_shape=jax.ShapeDtypeStruct((_B, _D), _f32),
        compiler_params=pltpu.CompilerParams(
            dimension_semantics=("arbitrary",),
            vmem_limit_bytes=120 * 1024 * 1024,
        ),
    )(tlo16, ntl16, lo16, hi16,
      oinv.reshape(1, _B), raw_state, W_ph1, W_pr1,
      b_ph1[:, None, :], b_ph2[:, None, :],
      b_pr1[:, None, :], b_pr2[:, None, :],
      ln_g[:, None, :], ln_b[:, None, :],
      W_ph2, W_pr2)

    out = pl.pallas_call(
        _stage_b_body,
        grid_spec=pltpu.PrefetchScalarGridSpec(
            num_scalar_prefetch=4,
            grid=(_NE,),
            in_specs=[_VMEM_FULL() for _ in range(4)] + [_HBM(), _HBM()],
            out_specs=_VMEM_FULL(),
            scratch_shapes=[
                pltpu.VMEM((_B, _D), _f32),
                pltpu.VMEM((2, _D, _H), _f32),
                pltpu.VMEM((2, _H, _D), _f32),
                pltpu.SemaphoreType.DMA((2, _NC)),
                pltpu.SemaphoreType.DMA((2, _NC)),
            ],
        ),
        out_shape=jax.ShapeDtypeStruct((_B, _D), _f32),
        compiler_params=pltpu.CompilerParams(
            dimension_semantics=("arbitrary",),
            vmem_limit_bytes=120 * 1024 * 1024,
        ),
    )(tlo8, ntl8, lo8, hi8,
      oinv.reshape(_B, 1), mixed_ln,
      b_t1[:, None, :], b_t2[:, None, :],
      W_t1, W_t2)

    return out[:, None, :]
